# Initial kernel scaffold; baseline (speedup 1.0000x reference)
import jax
import jax.numpy as jnp
from jax.experimental import pallas as pl


def kernel(x, edge_index, W1, b1, W2, b2, Wc, bc):
    raise NotImplementedError("write your pallas kernel here")


# speedup vs baseline: 26.7866x; 26.7866x over previous
import functools

import jax
import jax.numpy as jnp
from jax import lax
from jax.experimental import pallas as pl
from jax.experimental.pallas import tpu as pltpu
from jax.experimental.pallas import tpu_sc as plsc

N = 10000
NPAD = 10240
E = 320000
F = 32
NC = 2
NS = 16
NW = NC * NS
CHUNK = 128
CPW = 80
EPW = CPW * CHUNK
EPAD = NW * EPW
RPT = NPAD // NS


def _mesh():
    return plsc.VectorSubcoreMesh(core_axis_name="c", subcore_axis_name="s")


_SC_PARAMS = pltpu.CompilerParams(use_tc_tiling_on_sc=False)


def _sc_degree(dstg, ones):
    @functools.partial(
        pl.kernel,
        out_type=jax.ShapeDtypeStruct((NC, NPAD), jnp.float32),
        mesh=_mesh(),
        compiler_params=_SC_PARAMS,
        scratch_types=[
            pltpu.VMEM((CPW, CHUNK), jnp.int32),
            pltpu.VMEM((CHUNK,), jnp.float32),
            pltpu.VMEM_SHARED((NPAD,), jnp.float32),
            pltpu.SemaphoreType.DMA,
        ],
    )
    def k(dst_hbm, ones_hbm, out_hbm, dstv, onev, dacc, sem):
        c = lax.axis_index("c")
        s = lax.axis_index("s")
        wid = c * NS + s
        base = s * RPT
        pltpu.sync_copy(ones_hbm.at[pl.ds(base, RPT)], dacc.at[pl.ds(base, RPT)])
        pltpu.sync_copy(ones_hbm.at[pl.ds(0, CHUNK)], onev)
        pltpu.sync_copy(dst_hbm.at[wid], dstv)
        plsc.subcore_barrier()

        nburst = 8

        @pl.loop(0, CPW, step=nburst)
        def _(j):
            descs = []
            for t in range(nburst):
                descs.append(
                    pltpu.async_copy(onev, dacc.at[dstv.at[j + t]], sem, add=True)
                )
            for d in descs:
                d.wait()

        plsc.subcore_barrier()
        pltpu.sync_copy(dacc.at[pl.ds(base, RPT)], out_hbm.at[c, pl.ds(base, RPT)])

    return k(dstg, ones)


def _sc_edge(y, srcg, dstg):
    @functools.partial(
        pl.kernel,
        out_type=jax.ShapeDtypeStruct((NC, NPAD, F), jnp.float32),
        mesh=_mesh(),
        compiler_params=_SC_PARAMS,
        scratch_types=[
            pltpu.VMEM((CPW, CHUNK), jnp.int32),
            pltpu.VMEM((CPW, CHUNK), jnp.int32),
            pltpu.VMEM((2, CHUNK, F), jnp.float32),
            pltpu.VMEM_SHARED((NPAD, F), jnp.float32),
            pltpu.SemaphoreType.DMA,
            pltpu.SemaphoreType.DMA,
        ],
    )
    def k(y_hbm, src_hbm, dst_hbm, out_hbm, srcv, dstv, rows, acc, sem0, sem1):
        c = lax.axis_index("c")
        s = lax.axis_index("s")
        wid = c * NS + s
        base = s * RPT
        pltpu.sync_copy(src_hbm.at[wid], srcv)
        pltpu.sync_copy(dst_hbm.at[wid], dstv)
        pltpu.sync_copy(y_hbm.at[pl.ds(base, RPT)], acc.at[pl.ds(base, RPT)])
        plsc.subcore_barrier()

        sems = (sem0, sem1)
        for b in range(2):
            pltpu.async_copy(y_hbm.at[srcv.at[b]], rows.at[b], sems[b])

        @pl.loop(0, CPW, step=2)
        def _(j):
            for b in range(2):
                jj = j + b
                sem = sems[b]
                pltpu.make_async_copy(y_hbm.at[srcv.at[jj]], rows.at[b], sem).wait()
                pltpu.sync_copy(rows.at[b], acc.at[dstv.at[jj]], add=True)

                @pl.when(jj + 2 < CPW)
                def _():
                    pltpu.async_copy(y_hbm.at[srcv.at[jj + 2]], rows.at[b], sem)

        plsc.subcore_barrier()
        pltpu.sync_copy(acc.at[pl.ds(base, RPT)], out_hbm.at[c, pl.ds(base, RPT)])

    return k(y, srcg, dstg)


def _tc_matmul(x, w):
    def body(x_ref, w_ref, o_ref):
        o_ref[...] = jnp.dot(x_ref[...], w_ref[...], preferred_element_type=jnp.float32)

    return pl.pallas_call(
        body,
        out_shape=jax.ShapeDtypeStruct((x.shape[0], w.shape[1]), jnp.float32),
    )(x, w)


def _tc_scale(xw, d0, d1):
    def body(xw_ref, d0_ref, d1_ref, dis_ref, y_ref):
        dis = lax.rsqrt(d0_ref[...] + d1_ref[...] - 1.0)
        dis_ref[...] = dis
        y_ref[...] = xw_ref[...] * dis

    return pl.pallas_call(
        body,
        out_shape=(
            jax.ShapeDtypeStruct((NPAD, 1), jnp.float32),
            jax.ShapeDtypeStruct((NPAD, F), jnp.float32),
        ),
    )(xw, d0, d1)


def _tc_mid(p0, p1, y1, dis, b1, w2):
    def body(p0_ref, p1_ref, y1_ref, dis_ref, b1_ref, w2_ref, y2_ref):
        dis = dis_ref[...]
        h = dis * (p0_ref[...] + p1_ref[...] - y1_ref[...]) + b1_ref[...]
        h = jnp.maximum(h, 0.0)
        y2_ref[...] = jnp.dot(h, w2_ref[...], preferred_element_type=jnp.float32) * dis

    return pl.pallas_call(
        body,
        out_shape=jax.ShapeDtypeStruct((NPAD, F), jnp.float32),
    )(p0, p1, y1, dis, b1, w2)


def _tc_final(q0, q1, y2, dis, b2, wc, bc):
    def body(q0_ref, q1_ref, y2_ref, dis_ref, b2_ref, wc_ref, bc_ref, h2_ref, o_ref):
        h2 = dis_ref[...] * (q0_ref[...] + q1_ref[...] - y2_ref[...]) + b2_ref[...]
        h2_ref[...] = h2
        o_ref[...] = jnp.dot(h2, wc_ref[...], preferred_element_type=jnp.float32) + bc_ref[...]

    return pl.pallas_call(
        body,
        out_shape=(
            jax.ShapeDtypeStruct((NPAD, F), jnp.float32),
            jax.ShapeDtypeStruct((NPAD, 2), jnp.float32),
        ),
    )(q0, q1, y2, dis, b2, wc, bc)


def kernel(x, edge_index, W1, b1, W2, b2, Wc, bc):
    src = edge_index[0].astype(jnp.int32)
    dst = edge_index[1].astype(jnp.int32)
    npad_e = EPAD - E
    pad_src = jnp.zeros((npad_e,), jnp.int32)
    pad_dst = N + (jnp.arange(npad_e, dtype=jnp.int32) % (NPAD - N))
    srcg = jnp.concatenate([src, pad_src]).reshape(NW, CPW, CHUNK)
    dstg = jnp.concatenate([dst, pad_dst]).reshape(NW, CPW, CHUNK)
    x_pad = jnp.pad(x, ((0, NPAD - N), (0, 0)))
    ones = jnp.ones((NPAD,), jnp.float32)

    deg2 = _sc_degree(dstg, ones)
    xw1 = _tc_matmul(x_pad, W1)
    d0 = deg2[0].reshape(NPAD, 1)
    d1 = deg2[1].reshape(NPAD, 1)
    dis, y1 = _tc_scale(xw1, d0, d1)
    p = _sc_edge(y1, srcg, dstg)
    y2 = _tc_mid(p[0], p[1], y1, dis, b1[None, :], W2)
    q = _sc_edge(y2, srcg, dstg)
    h2, out = _tc_final(q[0], q[1], y2, dis, b2[None, :], Wc, bc[None, :])
    return (out[:N], h2[:N])


# speedup vs baseline: 45.6402x; 1.7038x over previous
import functools

import jax
import jax.numpy as jnp
from jax import lax
from jax.experimental import pallas as pl
from jax.experimental.pallas import tpu as pltpu
from jax.experimental.pallas import tpu_sc as plsc

N = 10000
NPAD = 10240
E = 320000
F = 32
NC = 2
NS = 16
CHUNK = 125
NCHUNKS = E // CHUNK
CPT0 = 104
CPT1 = (NCHUNKS - NS * CPT0) // NS
RPT = NPAD // NS
NBUF = 4


def _mesh():
    return plsc.VectorSubcoreMesh(core_axis_name="c", subcore_axis_name="s")


_SC_PARAMS = pltpu.CompilerParams(use_tc_tiling_on_sc=False)


def _chunk_base(c, s):
    return jnp.where(c == 0, s * CPT0, NS * CPT0 + s * CPT1)


def _sc_degree(dst2d, ones):
    @functools.partial(
        pl.kernel,
        out_type=jax.ShapeDtypeStruct((NC, NPAD), jnp.float32),
        mesh=_mesh(),
        compiler_params=_SC_PARAMS,
        scratch_types=[
            pltpu.VMEM((CPT0, CHUNK), jnp.int32),
            pltpu.VMEM((CHUNK,), jnp.float32),
            pltpu.VMEM_SHARED((NPAD,), jnp.float32),
            pltpu.SemaphoreType.DMA,
        ],
    )
    def k(dst_hbm, ones_hbm, out_hbm, dstv, onev, dacc, sem):
        c = lax.axis_index("c")
        s = lax.axis_index("s")
        base = s * RPT
        cbase = _chunk_base(c, s)
        pltpu.sync_copy(ones_hbm.at[pl.ds(base, RPT)], dacc.at[pl.ds(base, RPT)])
        pltpu.sync_copy(ones_hbm.at[pl.ds(0, CHUNK)], onev)

        def run(cpt):
            pltpu.sync_copy(dst_hbm.at[pl.ds(cbase, cpt)], dstv.at[pl.ds(0, cpt)])
            plsc.subcore_barrier()

            @pl.loop(0, cpt, step=8)
            def _(j):
                descs = []
                for t in range(8):
                    descs.append(
                        pltpu.async_copy(onev, dacc.at[dstv.at[j + t]], sem, add=True)
                    )
                for d in descs:
                    d.wait()

        @pl.when(c == 0)
        def _():
            run(CPT0)

        @pl.when(c != 0)
        def _():
            run(CPT1)

        plsc.subcore_barrier()
        pltpu.sync_copy(dacc.at[pl.ds(base, RPT)], out_hbm.at[c, pl.ds(base, RPT)])

    return k(dst2d, ones)


def _sc_edge(y, src2d, dst2d):
    @functools.partial(
        pl.kernel,
        out_type=jax.ShapeDtypeStruct((NC, NPAD, F), jnp.float32),
        mesh=_mesh(),
        compiler_params=_SC_PARAMS,
        scratch_types=[
            pltpu.VMEM((CPT0, CHUNK), jnp.int32),
            pltpu.VMEM((CPT0, CHUNK), jnp.int32),
            pltpu.VMEM((NBUF, CHUNK, F), jnp.float32),
            pltpu.VMEM_SHARED((NPAD, F), jnp.float32),
            [pltpu.SemaphoreType.DMA] * NBUF,
            [pltpu.SemaphoreType.DMA] * NBUF,
        ],
    )
    def k(y_hbm, src_hbm, dst_hbm, out_hbm, srcv, dstv, rows, acc, gsems, ssems):
        c = lax.axis_index("c")
        s = lax.axis_index("s")
        base = s * RPT
        cbase = _chunk_base(c, s)
        pltpu.sync_copy(y_hbm.at[pl.ds(base, RPT)], acc.at[pl.ds(base, RPT)])

        def run(cpt):
            pltpu.sync_copy(src_hbm.at[pl.ds(cbase, cpt)], srcv.at[pl.ds(0, cpt)])
            pltpu.sync_copy(dst_hbm.at[pl.ds(cbase, cpt)], dstv.at[pl.ds(0, cpt)])
            plsc.subcore_barrier()
            for b in range(NBUF):
                pltpu.async_copy(y_hbm.at[srcv.at[b]], rows.at[b], gsems[b])

            @pl.loop(0, cpt, step=NBUF)
            def _(j):
                for b in range(NBUF):
                    jj = j + b
                    pltpu.make_async_copy(
                        y_hbm.at[srcv.at[jj]], rows.at[b], gsems[b]
                    ).wait()
                    pltpu.async_copy(
                        rows.at[b], acc.at[dstv.at[jj]], ssems[b], add=True
                    )

                    @pl.when(jj + NBUF < cpt)
                    def _():
                        pltpu.make_async_copy(
                            rows.at[b], acc.at[dstv.at[jj]], ssems[b]
                        ).wait()
                        pltpu.async_copy(
                            y_hbm.at[srcv.at[jj + NBUF]], rows.at[b], gsems[b]
                        )

            for b in range(NBUF):
                pltpu.make_async_copy(
                    rows.at[b], acc.at[dstv.at[cpt - NBUF + b]], ssems[b]
                ).wait()

        @pl.when(c == 0)
        def _():
            run(CPT0)

        @pl.when(c != 0)
        def _():
            run(CPT1)

        plsc.subcore_barrier()
        pltpu.sync_copy(acc.at[pl.ds(base, RPT)], out_hbm.at[c, pl.ds(base, RPT)])

    return k(y, src2d, dst2d)


def _tc1(x, w1, d0, d1):
    def body(x_ref, w1_ref, d0_ref, d1_ref, dis_ref, y_ref):
        dis = lax.rsqrt(d0_ref[...] + d1_ref[...] - 1.0)
        dis_ref[...] = dis
        xw = jnp.dot(x_ref[...], w1_ref[...], preferred_element_type=jnp.float32)
        y_ref[...] = xw * dis

    return pl.pallas_call(
        body,
        out_shape=(
            jax.ShapeDtypeStruct((NPAD, 1), jnp.float32),
            jax.ShapeDtypeStruct((NPAD, F), jnp.float32),
        ),
    )(x, w1, d0, d1)


def _tc_mid(p0, p1, y1, dis, b1, w2):
    def body(p0_ref, p1_ref, y1_ref, dis_ref, b1_ref, w2_ref, y2_ref):
        dis = dis_ref[...]
        h = dis * (p0_ref[...] + p1_ref[...] - y1_ref[...]) + b1_ref[...]
        h = jnp.maximum(h, 0.0)
        y2_ref[...] = jnp.dot(h, w2_ref[...], preferred_element_type=jnp.float32) * dis

    return pl.pallas_call(
        body,
        out_shape=jax.ShapeDtypeStruct((NPAD, F), jnp.float32),
    )(p0, p1, y1, dis, b1, w2)


def _tc_final(q0, q1, y2, dis, b2, wc, bc):
    def body(q0_ref, q1_ref, y2_ref, dis_ref, b2_ref, wc_ref, bc_ref, h2_ref, o_ref):
        h2 = dis_ref[...] * (q0_ref[...] + q1_ref[...] - y2_ref[...]) + b2_ref[...]
        h2_ref[...] = h2[:N]
        o_ref[...] = (
            jnp.dot(h2[:N], wc_ref[...], preferred_element_type=jnp.float32)
            + bc_ref[...]
        )

    return pl.pallas_call(
        body,
        out_shape=(
            jax.ShapeDtypeStruct((N, F), jnp.float32),
            jax.ShapeDtypeStruct((N, 2), jnp.float32),
        ),
    )(q0, q1, y2, dis, b2, wc, bc)


def kernel(x, edge_index, W1, b1, W2, b2, Wc, bc):
    src2d = edge_index[0].astype(jnp.int32).reshape(NCHUNKS, CHUNK)
    dst2d = edge_index[1].astype(jnp.int32).reshape(NCHUNKS, CHUNK)
    x_pad = jnp.pad(x, ((0, NPAD - N), (0, 0)))
    ones = jnp.ones((NPAD,), jnp.float32)

    deg2 = _sc_degree(dst2d, ones)
    d0 = deg2[0].reshape(NPAD, 1)
    d1 = deg2[1].reshape(NPAD, 1)
    dis, y1 = _tc1(x_pad, W1, d0, d1)
    p = _sc_edge(y1, src2d, dst2d)
    y2 = _tc_mid(p[0], p[1], y1, dis, b1[None, :], W2)
    q = _sc_edge(y2, src2d, dst2d)
    h2, out = _tc_final(q[0], q[1], y2, dis, b2[None, :], Wc, bc[None, :])
    return (out, h2)


# speedup vs baseline: 54.3436x; 1.1907x over previous
import functools

import jax
import jax.numpy as jnp
from jax import lax
from jax.experimental import pallas as pl
from jax.experimental.pallas import tpu as pltpu
from jax.experimental.pallas import tpu_sc as plsc

N = 10000
NPAD = 10240
E = 320000
F = 32
NC = 2
NS = 16
CHUNK = 128
NCHUNKS = E // CHUNK
NBUF = 4
RPT = NPAD // NS
RPTF = N // NS

CPT0 = 84
_C1TOT = NCHUNKS - NS * CPT0
CPT1HI = _C1TOT // NS + 1
CPT1LO = _C1TOT // NS
NHI = _C1TOT - NS * CPT1LO
_CORE1_BASE = NS * CPT0


def _mesh():
    return plsc.VectorSubcoreMesh(core_axis_name="c", subcore_axis_name="s")


_SC_PARAMS = pltpu.CompilerParams(use_tc_tiling_on_sc=False)


def _sc_degree(e3, ones):
    @functools.partial(
        pl.kernel,
        out_type=jax.ShapeDtypeStruct((NC, NPAD, 1), jnp.float32),
        mesh=_mesh(),
        compiler_params=_SC_PARAMS,
        scratch_types=[
            pltpu.VMEM((CPT0, CHUNK), jnp.int32),
            pltpu.VMEM((CHUNK, 1), jnp.float32),
            pltpu.VMEM_SHARED((NPAD, 1), jnp.float32),
            pltpu.SemaphoreType.DMA,
        ],
    )
    def k(e_hbm, ones_hbm, out_hbm, dstv, onev, dacc, sem):
        c = lax.axis_index("c")
        s = lax.axis_index("s")
        base = s * RPT
        pltpu.sync_copy(ones_hbm.at[pl.ds(base, RPT)], dacc.at[pl.ds(base, RPT)])
        pltpu.sync_copy(ones_hbm.at[pl.ds(0, CHUNK)], onev)

        def run(cpt, cbase):
            pltpu.sync_copy(e_hbm.at[1, pl.ds(cbase, cpt)], dstv.at[pl.ds(0, cpt)])
            plsc.subcore_barrier()
            main = (cpt // NBUF) * NBUF

            @pl.loop(0, main, step=NBUF)
            def _(j):
                descs = []
                for t in range(NBUF):
                    descs.append(
                        pltpu.async_copy(onev, dacc.at[dstv.at[j + t]], sem, add=True)
                    )
                for d in descs:
                    d.wait()

            for jj in range(main, cpt):
                pltpu.sync_copy(onev, dacc.at[dstv.at[jj]], add=True)

        @pl.when(c == 0)
        def _():
            run(CPT0, s * CPT0)

        @pl.when(jnp.logical_and(c != 0, s < NHI))
        def _():
            run(CPT1HI, _CORE1_BASE + s * CPT1HI)

        @pl.when(jnp.logical_and(c != 0, s >= NHI))
        def _():
            run(CPT1LO, _CORE1_BASE + NHI * CPT1HI + (s - NHI) * CPT1LO)

        plsc.subcore_barrier()
        pltpu.sync_copy(dacc.at[pl.ds(base, RPT)], out_hbm.at[c, pl.ds(base, RPT)])

    return k(e3, ones)


def _sc_edge(y, e3):
    @functools.partial(
        pl.kernel,
        out_type=jax.ShapeDtypeStruct((NC, N, F), jnp.float32),
        mesh=_mesh(),
        compiler_params=_SC_PARAMS,
        scratch_types=[
            pltpu.VMEM((CPT0, CHUNK), jnp.int32),
            pltpu.VMEM((CPT0, CHUNK), jnp.int32),
            pltpu.VMEM((NBUF, CHUNK, F), jnp.float32),
            pltpu.VMEM_SHARED((N, F), jnp.float32),
            [pltpu.SemaphoreType.DMA] * NBUF,
            [pltpu.SemaphoreType.DMA] * NBUF,
        ],
    )
    def k(y_hbm, e_hbm, out_hbm, srcv, dstv, rows, acc, gsems, ssems):
        c = lax.axis_index("c")
        s = lax.axis_index("s")
        base = s * RPTF
        pltpu.sync_copy(y_hbm.at[pl.ds(base, RPTF)], acc.at[pl.ds(base, RPTF)])

        def run(cpt, cbase):
            pltpu.sync_copy(e_hbm.at[0, pl.ds(cbase, cpt)], srcv.at[pl.ds(0, cpt)])
            pltpu.sync_copy(e_hbm.at[1, pl.ds(cbase, cpt)], dstv.at[pl.ds(0, cpt)])
            plsc.subcore_barrier()
            for b in range(NBUF):
                pltpu.async_copy(y_hbm.at[srcv.at[b]], rows.at[b], gsems[b])
            main = (cpt // NBUF) * NBUF

            @pl.loop(0, main, step=NBUF)
            def _(j):
                for b in range(NBUF):
                    jj = j + b
                    pltpu.make_async_copy(
                        y_hbm.at[srcv.at[jj]], rows.at[b], gsems[b]
                    ).wait()
                    pltpu.async_copy(
                        rows.at[b], acc.at[dstv.at[jj]], ssems[b], add=True
                    )

                    @pl.when(jj + NBUF < cpt)
                    def _():
                        pltpu.make_async_copy(
                            rows.at[b], acc.at[dstv.at[jj]], ssems[b]
                        ).wait()
                        pltpu.async_copy(
                            y_hbm.at[srcv.at[jj + NBUF]], rows.at[b], gsems[b]
                        )

            for jj in range(main, cpt):
                b = jj % NBUF
                pltpu.make_async_copy(y_hbm.at[srcv.at[jj]], rows.at[b], gsems[b]).wait()
                pltpu.async_copy(rows.at[b], acc.at[dstv.at[jj]], ssems[b], add=True)
            for jj in range(max(0, cpt - NBUF), cpt):
                b = jj % NBUF
                pltpu.make_async_copy(rows.at[b], acc.at[dstv.at[jj]], ssems[b]).wait()

        @pl.when(c == 0)
        def _():
            run(CPT0, s * CPT0)

        @pl.when(jnp.logical_and(c != 0, s < NHI))
        def _():
            run(CPT1HI, _CORE1_BASE + s * CPT1HI)

        @pl.when(jnp.logical_and(c != 0, s >= NHI))
        def _():
            run(CPT1LO, _CORE1_BASE + NHI * CPT1HI + (s - NHI) * CPT1LO)

        plsc.subcore_barrier()
        pltpu.sync_copy(acc.at[pl.ds(base, RPTF)], out_hbm.at[c, pl.ds(base, RPTF)])

    return k(y, e3)


def _tc1(x, w1, deg2):
    def body(x_ref, w1_ref, deg_ref, dis_ref, y_ref):
        d = deg_ref[0] + deg_ref[1] - 1.0
        dis = jnp.broadcast_to(lax.rsqrt(d)[:N], (N, F))
        dis_ref[...] = dis
        xw = jnp.dot(x_ref[...], w1_ref[...], preferred_element_type=jnp.float32)
        y_ref[...] = xw * dis

    return pl.pallas_call(
        body,
        out_shape=(
            jax.ShapeDtypeStruct((N, F), jnp.float32),
            jax.ShapeDtypeStruct((N, F), jnp.float32),
        ),
    )(x, w1, deg2)


def _tc_mid(p, y1, dis, b1, w2):
    def body(p_ref, y1_ref, dis_ref, b1_ref, w2_ref, y2_ref):
        dis = dis_ref[...]
        h = dis * (p_ref[0] + p_ref[1] - y1_ref[...]) + b1_ref[...]
        h = jnp.maximum(h, 0.0)
        y2_ref[...] = jnp.dot(h, w2_ref[...], preferred_element_type=jnp.float32) * dis

    return pl.pallas_call(
        body,
        out_shape=jax.ShapeDtypeStruct((N, F), jnp.float32),
    )(p, y1, dis, b1, w2)


def _tc_final(q, y2, dis, b2, wc, bc):
    def body(q_ref, y2_ref, dis_ref, b2_ref, wc_ref, bc_ref, h2_ref, o_ref):
        h2 = dis_ref[...] * (q_ref[0] + q_ref[1] - y2_ref[...]) + b2_ref[...]
        h2_ref[...] = h2
        o_ref[...] = (
            jnp.dot(h2, wc_ref[...], preferred_element_type=jnp.float32) + bc_ref[...]
        )

    return pl.pallas_call(
        body,
        out_shape=(
            jax.ShapeDtypeStruct((N, F), jnp.float32),
            jax.ShapeDtypeStruct((N, 2), jnp.float32),
        ),
    )(q, y2, dis, b2, wc, bc)


def kernel(x, edge_index, W1, b1, W2, b2, Wc, bc):
    e3 = edge_index.astype(jnp.int32).reshape(2, NCHUNKS, CHUNK)
    ones = jnp.ones((NPAD, 1), jnp.float32)

    deg2 = _sc_degree(e3, ones)
    dis, y1 = _tc1(x, W1, deg2)
    p = _sc_edge(y1, e3)
    y2 = _tc_mid(p, y1, dis, b1, W2)
    q = _sc_edge(y2, e3)
    h2, out = _tc_final(q, y2, dis, b2, Wc, bc)
    return (out, h2)


# speedup vs baseline: 64.6528x; 1.1897x over previous
import functools

import jax
import jax.numpy as jnp
from jax import lax
from jax.experimental import pallas as pl
from jax.experimental.pallas import tpu as pltpu
from jax.experimental.pallas import tpu_sc as plsc

N = 10000
E = 320000
F = 32
NROW = N * F // 128
NC = 2
NS = 16
CHUNK = 128
NCHUNKS = E // CHUNK
NBUF = 4
RPTF = N // NS

CPT0 = 84
_C1TOT = NCHUNKS - NS * CPT0
CPT1HI = _C1TOT // NS + 1
CPT1LO = _C1TOT // NS
NHI = _C1TOT - NS * CPT1LO
_CORE1_BASE = NS * CPT0


def _mesh():
    return plsc.VectorSubcoreMesh(core_axis_name="c", subcore_axis_name="s")


_SC_PARAMS = pltpu.CompilerParams(use_tc_tiling_on_sc=False)


def _per_tile(c, s, run):
    @pl.when(c == 0)
    def _():
        run(CPT0, s * CPT0)

    @pl.when(jnp.logical_and(c != 0, s < NHI))
    def _():
        run(CPT1HI, _CORE1_BASE + s * CPT1HI)

    @pl.when(jnp.logical_and(c != 0, s >= NHI))
    def _():
        run(CPT1LO, _CORE1_BASE + NHI * CPT1HI + (s - NHI) * CPT1LO)


def _sc_degree(e3, ones):
    @functools.partial(
        pl.kernel,
        out_type=jax.ShapeDtypeStruct((NC, N, F), jnp.float32),
        mesh=_mesh(),
        compiler_params=_SC_PARAMS,
        scratch_types=[
            pltpu.VMEM((CPT0, CHUNK), jnp.int32),
            pltpu.VMEM((CHUNK, F), jnp.float32),
            pltpu.VMEM_SHARED((N, F), jnp.float32),
            pltpu.SemaphoreType.DMA,
        ],
    )
    def k(e_hbm, ones_hbm, out_hbm, dstv, onev, dacc, sem):
        c = lax.axis_index("c")
        s = lax.axis_index("s")
        base = s * RPTF
        pltpu.sync_copy(ones_hbm.at[pl.ds(base, RPTF)], dacc.at[pl.ds(base, RPTF)])
        pltpu.sync_copy(ones_hbm.at[pl.ds(0, CHUNK)], onev)

        def run(cpt, cbase):
            pltpu.sync_copy(e_hbm.at[1, pl.ds(cbase, cpt)], dstv.at[pl.ds(0, cpt)])
            plsc.subcore_barrier()
            main = (cpt // NBUF) * NBUF

            @pl.loop(0, main, step=NBUF)
            def _(j):
                descs = []
                for t in range(NBUF):
                    descs.append(
                        pltpu.async_copy(onev, dacc.at[dstv.at[j + t]], sem, add=True)
                    )
                for d in descs:
                    d.wait()

            for jj in range(main, cpt):
                pltpu.sync_copy(onev, dacc.at[dstv.at[jj]], add=True)

        _per_tile(c, s, run)
        plsc.subcore_barrier()
        pltpu.sync_copy(dacc.at[pl.ds(base, RPTF)], out_hbm.at[c, pl.ds(base, RPTF)])

    return k(e3, ones)


def _sc_edge(y, e3):
    @functools.partial(
        pl.kernel,
        out_type=jax.ShapeDtypeStruct((NC, N, F), jnp.float32),
        mesh=_mesh(),
        compiler_params=_SC_PARAMS,
        scratch_types=[
            pltpu.VMEM((CPT0, CHUNK), jnp.int32),
            pltpu.VMEM((CPT0, CHUNK), jnp.int32),
            pltpu.VMEM((NBUF, CHUNK, F), jnp.float32),
            pltpu.VMEM_SHARED((N, F), jnp.float32),
            [pltpu.SemaphoreType.DMA] * NBUF,
            [pltpu.SemaphoreType.DMA] * NBUF,
        ],
    )
    def k(y_hbm, e_hbm, out_hbm, srcv, dstv, rows, acc, gsems, ssems):
        c = lax.axis_index("c")
        s = lax.axis_index("s")
        base = s * RPTF
        pltpu.sync_copy(y_hbm.at[pl.ds(base, RPTF)], acc.at[pl.ds(base, RPTF)])

        def run(cpt, cbase):
            pltpu.sync_copy(e_hbm.at[0, pl.ds(cbase, cpt)], srcv.at[pl.ds(0, cpt)])
            pltpu.sync_copy(e_hbm.at[1, pl.ds(cbase, cpt)], dstv.at[pl.ds(0, cpt)])
            plsc.subcore_barrier()
            for b in range(NBUF):
                pltpu.async_copy(y_hbm.at[srcv.at[b]], rows.at[b], gsems[b])
            main = (cpt // NBUF) * NBUF

            @pl.loop(0, main, step=NBUF)
            def _(j):
                for b in range(NBUF):
                    jj = j + b
                    pltpu.make_async_copy(
                        y_hbm.at[srcv.at[jj]], rows.at[b], gsems[b]
                    ).wait()
                    pltpu.async_copy(
                        rows.at[b], acc.at[dstv.at[jj]], ssems[b], add=True
                    )

                    @pl.when(jj + NBUF < cpt)
                    def _():
                        pltpu.make_async_copy(
                            rows.at[b], acc.at[dstv.at[jj]], ssems[b]
                        ).wait()
                        pltpu.async_copy(
                            y_hbm.at[srcv.at[jj + NBUF]], rows.at[b], gsems[b]
                        )

            for jj in range(main, cpt):
                b = jj % NBUF
                pltpu.make_async_copy(y_hbm.at[srcv.at[jj]], rows.at[b], gsems[b]).wait()
                pltpu.async_copy(rows.at[b], acc.at[dstv.at[jj]], ssems[b], add=True)
            for jj in range(max(0, cpt - NBUF), cpt):
                b = jj % NBUF
                pltpu.make_async_copy(rows.at[b], acc.at[dstv.at[jj]], ssems[b]).wait()

        _per_tile(c, s, run)
        plsc.subcore_barrier()
        pltpu.sync_copy(acc.at[pl.ds(base, RPTF)], out_hbm.at[c, pl.ds(base, RPTF)])

    return k(y, e3)


def _blockdiag(w_ref, out_ref, copies):
    kk, nn = w_ref.shape
    out_ref[...] = jnp.zeros(out_ref.shape, jnp.float32)
    for j in range(copies):
        out_ref[pl.ds(j * kk, kk), pl.ds(j * nn, nn)] = w_ref[...]


def _tc1(x, w1, degx):
    def body(x_ref, w1_ref, deg_ref, dis_ref, y_ref, w1b):
        _blockdiag(w1_ref, w1b, 4)
        dis = lax.rsqrt(deg_ref[0] + deg_ref[1] - 1.0)
        dis_ref[...] = dis
        x4 = jnp.reshape(x_ref[...], (NROW, 512))
        xw = jnp.dot(x4, w1b[...], preferred_element_type=jnp.float32)
        y_ref[...] = xw * dis

    return pl.pallas_call(
        body,
        out_shape=(
            jax.ShapeDtypeStruct((NROW, 128), jnp.float32),
            jax.ShapeDtypeStruct((NROW, 128), jnp.float32),
        ),
        scratch_shapes=[pltpu.VMEM((512, 128), jnp.float32)],
    )(x, w1, degx)


def _tc_mid(px, y1x, dis, b1, w2):
    def body(p_ref, y1_ref, dis_ref, b1_ref, w2_ref, y2_ref, w2b):
        _blockdiag(w2_ref, w2b, 4)
        dis = dis_ref[...]
        b128 = jnp.concatenate([b1_ref[...]] * 4)
        h = dis * (p_ref[0] + p_ref[1] - y1_ref[...]) + b128
        h = jnp.maximum(h, 0.0)
        y2_ref[...] = jnp.dot(h, w2b[...], preferred_element_type=jnp.float32) * dis

    return pl.pallas_call(
        body,
        out_shape=jax.ShapeDtypeStruct((NROW, 128), jnp.float32),
        scratch_shapes=[pltpu.VMEM((128, 128), jnp.float32)],
    )(px, y1x, dis, b1, w2)


def _tc_final(qx, y2x, dis, b2, wc, bc):
    def body(q_ref, y2_ref, dis_ref, b2_ref, wc_ref, bc_ref, h2_ref, o_ref, wcb):
        _blockdiag(wc_ref, wcb, 4)
        b128 = jnp.concatenate([b2_ref[...]] * 4)
        h2x = dis_ref[...] * (q_ref[0] + q_ref[1] - y2_ref[...]) + b128
        h2_ref[...] = h2x
        bc8 = jnp.concatenate([bc_ref[...]] * 4)
        o_ref[...] = (
            jnp.dot(h2x, wcb[...], preferred_element_type=jnp.float32) + bc8
        )

    return pl.pallas_call(
        body,
        out_shape=(
            jax.ShapeDtypeStruct((NROW, 128), jnp.float32),
            jax.ShapeDtypeStruct((NROW, 8), jnp.float32),
        ),
        scratch_shapes=[pltpu.VMEM((128, 8), jnp.float32)],
    )(qx, y2x, dis, b2, wc, bc)


def kernel(x, edge_index, W1, b1, W2, b2, Wc, bc):
    e3 = edge_index.astype(jnp.int32).reshape(2, NCHUNKS, CHUNK)
    ones = jnp.ones((NROW, 128), jnp.float32).reshape(N, F)

    deg2 = _sc_degree(e3, ones)
    degx = deg2.reshape(NC, NROW, 128)
    dis, y1x = _tc1(x, W1, degx)
    p = _sc_edge(y1x.reshape(N, F), e3)
    y2x = _tc_mid(p.reshape(NC, NROW, 128), y1x, dis, b1, W2)
    q = _sc_edge(y2x.reshape(N, F), e3)
    h2x, outx = _tc_final(q.reshape(NC, NROW, 128), y2x, dis, b2, Wc, bc)
    return (outx.reshape(N, 2), h2x.reshape(N, F))
